# Initial kernel scaffold; baseline (speedup 1.0000x reference)
import jax
import jax.numpy as jnp
from jax.experimental import pallas as pl


def kernel(x, batch):
    raise NotImplementedError("write your pallas kernel here")


# speedup vs baseline: 3.2713x; 3.2713x over previous
import functools

import jax
import jax.numpy as jnp
from jax import lax
from jax.experimental import pallas as pl
from jax.experimental.pallas import tpu as pltpu
from jax.experimental.pallas import tpu_sc as plsc

_N = 100000
_G = 128
_D = 512
_NW = 32
_CHUNK = 3136
_LAST = _N - _CHUNK * (_NW - 1)
_PAD_N = 100352
_SEARCH_STEPS = 17

_mesh = plsc.VectorSubcoreMesh(core_axis_name="c", subcore_axis_name="s")


@functools.partial(
    pl.kernel,
    mesh=_mesh,
    out_type=jax.ShapeDtypeStruct((_N,), jnp.float32),
    scratch_types=[
        pltpu.VMEM((_PAD_N,), jnp.int32),
        pltpu.VMEM((_CHUNK,), jnp.float32),
    ],
)
def _sc_node_degree(batch_hbm, nodedeg_hbm, batch_v, chunk_v):
    wid = lax.axis_index("s") * 2 + lax.axis_index("c")
    base = wid * _CHUNK
    n_nodes = jnp.where(wid == _NW - 1, _LAST, _CHUNK)
    wend = base + n_nodes

    pltpu.sync_copy(batch_hbm, batch_v.at[pl.ds(0, _N)])

    lane = lax.iota(jnp.int32, 16)

    def load_at(idx):
        vv = batch_v[pl.ds(idx, 16)]
        return vv[0]

    def lower_bound(tgt):
        def step(i, lo):
            s = jnp.int32(1 << (_SEARCH_STEPS - 1)) >> i
            cand = lo + s
            idx = jnp.minimum(cand, _N) - 1
            v = load_at(idx)
            ok = (cand <= _N) & (v < tgt)
            return jnp.where(ok, cand, lo)

        return lax.fori_loop(0, _SEARCH_STEPS, step, jnp.int32(0))

    first = load_at(base)
    last = load_at(wend - 1)

    def seg_step(g, prev_lb):
        nxt = lower_bound(g + 1)
        degf = (nxt - prev_lb).astype(jnp.float32)
        s = jnp.maximum(prev_lb, base) - base
        e = jnp.minimum(nxt, wend) - base
        j0 = s // 16
        j1 = (e + 15) // 16

        def fill(j, carry):
            p = 16 * j + lane
            old = chunk_v[pl.ds(16 * j, 16)]
            m = (p >= s) & (p < e)
            chunk_v[pl.ds(16 * j, 16)] = jnp.where(m, degf, old)
            return carry

        lax.fori_loop(j0, j1, fill, jnp.int32(0))
        return nxt

    lax.fori_loop(first, last + 1, seg_step, lower_bound(first))

    @pl.when(wid < _NW - 1)
    def _():
        pltpu.sync_copy(chunk_v, nodedeg_hbm.at[pl.ds(base, _CHUNK)])

    @pl.when(wid == _NW - 1)
    def _():
        pltpu.sync_copy(
            chunk_v.at[pl.ds(0, _LAST)], nodedeg_hbm.at[pl.ds(base, _LAST)]
        )


def _tc_scale_body(x_ref, d_ref, o_ref):
    o_ref[...] = x_ref[...] * lax.rsqrt(d_ref[...])


_ROWS_PER_BLOCK = 2000
_N_BLOCKS = _N // _ROWS_PER_BLOCK

_tc_scale = pl.pallas_call(
    _tc_scale_body,
    grid=(_N_BLOCKS,),
    in_specs=[
        pl.BlockSpec((_ROWS_PER_BLOCK, _D), lambda i: (i, 0)),
        pl.BlockSpec((_ROWS_PER_BLOCK, 1), lambda i: (i, 0)),
    ],
    out_specs=pl.BlockSpec((_ROWS_PER_BLOCK, _D), lambda i: (i, 0)),
    out_shape=jax.ShapeDtypeStruct((_N, _D), jnp.float32),
)


@jax.jit
def kernel(x, batch):
    node_deg = _sc_node_degree(batch.astype(jnp.int32))
    return _tc_scale(x, node_deg.reshape(_N, 1))
